# Initial kernel scaffold; baseline (speedup 1.0000x reference)
import jax
import jax.numpy as jnp
from jax.experimental import pallas as pl


def kernel(h_V, h_E, E_idx, mask_V, mask_attend, W1_w, W1_b, W2_w, W2_b, W3_w, W3_b, W11_w, W11_b, W12_w, W12_b, W13_w, W13_b, Win_w, Win_b, Wout_w, Wout_b, ln1_g, ln1_b, ln2_g, ln2_b, ln3_g, ln3_b):
    raise NotImplementedError("write your pallas kernel here")


# speedup vs baseline: 1358.2137x; 1358.2137x over previous
import functools

import jax
import jax.numpy as jnp
from jax import lax
from jax.experimental import pallas as pl
from jax.experimental.pallas import tpu as pltpu
from jax.experimental.pallas import tpu_sc as plsc

B, N, K, C = 2, 2048, 48, 128
BN = B * N
E = BN * K
SCALE = 30.0
TN = 128
TNK = TN * K

NC, NS = 2, 16
NW = NC * NS
PW = E // NW
CH = 128
NCH = PW // CH

_pallas_call = pl.pallas_call


def _gelu(x):
    return 0.5 * x * (1.0 + lax.erf(x * 0.7071067811865476))


def _ln(x, g, b):
    mu = jnp.mean(x, axis=-1, keepdims=True)
    xc = x - mu
    var = jnp.mean(xc * xc, axis=-1, keepdims=True)
    return xc * lax.rsqrt(var + 1e-5) * g + b


def _dot(a, b):
    return jnp.dot(a, b, preferred_element_type=jnp.float32)


def _prep_body(hv, w1a, b1, w1c, a1_out, c1_out):
    hv_ = hv[...]
    a1_out[...] = _dot(hv_, w1a[...]) + b1[...]
    c1_out[...] = _dot(hv_, w1c[...])


def _prep(hv2, w1a, b1r, w1c):
    return _pallas_call(
        _prep_body,
        out_shape=[
            jax.ShapeDtypeStruct((BN, C), jnp.float32),
            jax.ShapeDtypeStruct((BN, C), jnp.float32),
        ],
    )(hv2, w1a, b1r, w1c)


def _sc_gather(table, idx3):
    mesh = plsc.VectorSubcoreMesh(core_axis_name="c", subcore_axis_name="s")

    @functools.partial(
        pl.kernel,
        mesh=mesh,
        out_type=jax.ShapeDtypeStruct((E, C), jnp.float32),
        scratch_types=[
            pltpu.VMEM((NCH, CH), jnp.int32),
            pltpu.VMEM((CH, C), jnp.float32),
            pltpu.SemaphoreType.DMA,
        ],
    )
    def k(table_hbm, idx_hbm, out_hbm, idx_v, rows_v, gsem):
        wid = lax.axis_index("s") * NC + lax.axis_index("c")
        pltpu.sync_copy(idx_hbm.at[wid], idx_v)
        base = wid * PW

        def body(j, carry):
            pltpu.async_copy(table_hbm.at[idx_v.at[j]], rows_v, gsem).wait()
            pltpu.sync_copy(rows_v, out_hbm.at[pl.ds(base + j * CH, CH)])
            return carry

        lax.fori_loop(0, NCH, body, 0)

    return k(table, idx3)


_gather_impl = _sc_gather


def _tc1_body(hv, a1, he, g1, mav, mv,
              w1b, w2, b2, w3, b3, win, bin_, wout, bout,
              l1g, l1b, l2g, l2b, w11a, b11, w11c,
              hv_out, a2_out, c2_out):
    x = _dot(he[...], w1b[...]) + g1[...]
    x = (x.reshape(TN, K, C) + a1[...][:, None, :]).reshape(TNK, C)
    m = _gelu(x)
    m = _gelu(_dot(m, w2[...]) + b2[...])
    m = _dot(m, w3[...]) + b3[...]
    m = m.reshape(TN, K, C) * mav[...][:, :, None]
    dh = jnp.sum(m, axis=1) * (1.0 / SCALE)
    v = _ln(hv[...] + dh, l1g[...], l1b[...])
    f = _dot(_gelu(_dot(v, win[...]) + bin_[...]), wout[...]) + bout[...]
    v2 = _ln(v + f, l2g[...], l2b[...]) * mv[...]
    hv_out[...] = v2
    a2_out[...] = _dot(v2, w11a[...]) + b11[...]
    c2_out[...] = _dot(v2, w11c[...])


def _tc1(hv2, a1, he2, g1, mav2, mv2, w1b, w2, b2, w3, b3,
         win, binr, wout, boutr, l1g, l1b, l2g, l2b, w11a, b11, w11c):
    grid = (BN // TN,)
    node = pl.BlockSpec((TN, C), lambda i: (i, 0))
    edge = pl.BlockSpec((TNK, C), lambda i: (i, 0))
    full = lambda s: pl.BlockSpec(s, lambda i: (0,) * len(s))
    return _pallas_call(
        _tc1_body,
        grid=grid,
        in_specs=[
            node, node, edge, edge,
            pl.BlockSpec((TN, K), lambda i: (i, 0)),
            pl.BlockSpec((TN, 1), lambda i: (i, 0)),
            full((C, C)), full((C, C)), full((1, C)), full((C, C)), full((1, C)),
            full((C, 4 * C)), full((1, 4 * C)), full((4 * C, C)), full((1, C)),
            full((1, C)), full((1, C)), full((1, C)), full((1, C)),
            full((C, C)), full((1, C)), full((C, C)),
        ],
        out_specs=[node, node, node],
        out_shape=[
            jax.ShapeDtypeStruct((BN, C), jnp.float32),
            jax.ShapeDtypeStruct((BN, C), jnp.float32),
            jax.ShapeDtypeStruct((BN, C), jnp.float32),
        ],
    )(hv2, a1, he2, g1, mav2, mv2, w1b, w2, b2, w3, b3,
      win, binr, wout, boutr, l1g, l1b, l2g, l2b, w11a, b11, w11c)


def _tc2_body(a2, he, g2, mav, w11b, w12, b12, w13, b13, l3g, l3b, he_out):
    x = _dot(he[...], w11b[...]) + g2[...]
    x = (x.reshape(TN, K, C) + a2[...][:, None, :]).reshape(TNK, C)
    m = _gelu(x)
    m = _gelu(_dot(m, w12[...]) + b12[...])
    m = _dot(m, w13[...]) + b13[...]
    m = m.reshape(TN, K, C) * mav[...][:, :, None]
    e = _ln(he[...].reshape(TN, K, C) + m, l3g[...], l3b[...])
    he_out[...] = e.reshape(TNK, C)


def _tc2(a2, he2, g2, mav2, w11b, w12, b12, w13, b13, l3g, l3b):
    grid = (BN // TN,)
    node = pl.BlockSpec((TN, C), lambda i: (i, 0))
    edge = pl.BlockSpec((TNK, C), lambda i: (i, 0))
    full = lambda s: pl.BlockSpec(s, lambda i: (0,) * len(s))
    return _pallas_call(
        _tc2_body,
        grid=grid,
        in_specs=[
            node, edge, edge,
            pl.BlockSpec((TN, K), lambda i: (i, 0)),
            full((C, C)), full((C, C)), full((1, C)), full((C, C)), full((1, C)),
            full((1, C)), full((1, C)),
        ],
        out_specs=[edge],
        out_shape=[jax.ShapeDtypeStruct((E, C), jnp.float32)],
    )(a2, he2, g2, mav2, w11b, w12, b12, w13, b13, l3g, l3b)[0]


def kernel(h_V, h_E, E_idx, mask_V, mask_attend,
           W1_w, W1_b, W2_w, W2_b, W3_w, W3_b,
           W11_w, W11_b, W12_w, W12_b, W13_w, W13_b,
           Win_w, Win_b, Wout_w, Wout_b,
           ln1_g, ln1_b, ln2_g, ln2_b, ln3_g, ln3_b):
    hv2 = h_V.reshape(BN, C)
    he2 = h_E.reshape(E, C)
    offs = (jnp.arange(B, dtype=jnp.int32) * N)[:, None, None]
    idx3 = (E_idx + offs).reshape(NW, NCH, CH)
    mav2 = mask_attend.reshape(BN, K)
    mv2 = mask_V.reshape(BN, 1)

    w1a, w1b, w1c = W1_w[:C], W1_w[C:2 * C], W1_w[2 * C:]
    w11a, w11b, w11c = W11_w[:C], W11_w[C:2 * C], W11_w[2 * C:]
    r = lambda v: v.reshape(1, -1)

    a1, c1 = _prep(hv2, w1a, r(W1_b), w1c)
    g1 = _gather_impl(c1, idx3)
    hv_out, a2, c2 = _tc1(
        hv2, a1, he2, g1, mav2, mv2,
        w1b, W2_w, r(W2_b), W3_w, r(W3_b),
        Win_w, r(Win_b), Wout_w, r(Wout_b),
        r(ln1_g), r(ln1_b), r(ln2_g), r(ln2_b),
        w11a, r(W11_b), w11c)
    g2 = _gather_impl(c2, idx3)
    he_out = _tc2(a2, he2, g2, mav2,
                  w11b, W12_w, r(W12_b), W13_w, r(W13_b),
                  r(ln3_g), r(ln3_b))
    return hv_out.reshape(B, N, C), he_out.reshape(B, N, K, C)
